# baseline (speedup 1.0000x reference)
import jax
import jax.numpy as jnp
from jax.experimental import pallas as pl
from jax.experimental.pallas import tpu as pltpu

N = 4096
B = 8
UNITS = 16
BU = B * UNITS
M = 3
CPAD = 32
F = B * CPAD
BLK = 512
NJB = N // BLK
BT = 256
NJT = N // BT
PBLK = 512
NP = N // PBLK
NSTEPS = NJT * NJT + 1


def _h2n_body(h_ref, o_ref):
    o_ref[...] = jnp.concatenate([h_ref[b] for b in range(B)], axis=1)


def _h2n(h_bnu):
    return pl.pallas_call(
        _h2n_body,
        grid=(NJB,),
        in_specs=[pl.BlockSpec((B, BLK, UNITS), lambda j: (0, j, 0))],
        out_specs=pl.BlockSpec((BLK, BU), lambda j: (j, 0)),
        out_shape=jax.ShapeDtypeStruct((N, BU), jnp.float32),
    )(h_bnu)


def _n2b_body(x_ref, o_ref):
    for b in range(B):
        o_ref[b] = x_ref[:, b * UNITS:(b + 1) * UNITS]


def _n2b(x_n):
    return pl.pallas_call(
        _n2b_body,
        grid=(NJB,),
        in_specs=[pl.BlockSpec((BLK, BU), lambda j: (j, 0))],
        out_specs=pl.BlockSpec((B, BLK, UNITS), lambda j: (0, j, 0)),
        out_shape=jax.ShapeDtypeStruct((B, N, UNITS), jnp.float32),
    )(x_n)


def _mega_body(a_ref, at_ref, inp_ref, hx0_ref, hx1_ref,
               w0_ref, w12_ref, bias_ref,
               nh0_ref, nh1_ref,
               amax_s, dis_s, accd, x0s, x1b_s, xs_s, rh_s, u_s):
    t = pl.program_id(0)

    @pl.when(t < NJT * NJT)
    def _():
        i = t // NJT
        j = t % NJT
        m = jnp.maximum(a_ref[...], at_ref[...].T)
        amax_s[j, pl.ds(i * BT, BT), :] = m.astype(jnp.bfloat16)

        @pl.when(j == 0)
        def _():
            accd[...] = jnp.zeros_like(accd)

        accd[...] += jnp.sum(m, axis=1, keepdims=True).T

        @pl.when(j == NJT - 1)
        def _():
            d = accd[...]
            dis_s[i] = jnp.where(
                d > 0, jax.lax.rsqrt(jnp.maximum(d, 1e-12)), 0.0)

    @pl.when(t == NJT * NJT)
    def _():
        def dcol(i):
            return jnp.concatenate(
                [dis_s[2 * i].T, dis_s[2 * i + 1].T], axis=0)

        def panel_dot(sl):
            def jbody(j, acc):
                return acc + jnp.dot(
                    amax_s[j, sl, :], xs_s[pl.ds(j * BT, BT), :],
                    preferred_element_type=jnp.float32)
            return jax.lax.fori_loop(
                0, NJT, jbody, jnp.zeros((PBLK, F), jnp.float32))

        def assemble(get_cur, get_h, ci):
            pad = CPAD - ci - UNITS

            def body(i, _):
                sl = pl.ds(i * BLK, BLK)
                curb = get_cur(sl)
                hb = get_h(sl)
                pieces = []
                for b in range(B):
                    sub = [curb[:, b * ci:(b + 1) * ci],
                           hb[:, b * UNITS:(b + 1) * UNITS]]
                    if pad:
                        sub.append(jnp.zeros((BLK, pad), jnp.float32))
                    pieces.append(jnp.concatenate(sub, axis=1))
                x0s[sl, :] = jnp.concatenate(pieces, axis=1)
                return 0

            jax.lax.fori_loop(0, NJB, body, 0)

        def scale_to_xs(src):
            def body(i, _):
                sl = pl.ds(i * BLK, BLK)
                xs_s[sl, :] = (src[sl, :] * dcol(i)).astype(jnp.bfloat16)
                return 0

            jax.lax.fori_loop(0, NJB, body, 0)

        def gconv(get_cur, get_h, ci, wi, gate):
            assemble(get_cur, get_h, ci)
            scale_to_xs(x0s)

            def x1_body(p, _):
                sl = pl.ds(p * PBLK, PBLK)
                x1b_s[sl, :] = (-dcol(p) * panel_dot(sl)
                                ).astype(jnp.bfloat16)
                return 0

            jax.lax.fori_loop(0, NP, x1_body, 0)
            scale_to_xs(x1b_s)

            def x2_body(p, _):
                sl = pl.ds(p * PBLK, PBLK)
                x2v = -2.0 * dcol(p) * panel_dot(sl) - x0s[sl, :]
                x2b = x2v.astype(jnp.bfloat16)

                def cmb(k):
                    acc = bias_ref[k][None, :]
                    acc = acc + jnp.dot(x0s[sl, :], w0_ref[k],
                                        preferred_element_type=jnp.float32)
                    acc = acc + jnp.dot(x1b_s[sl, :], w12_ref[k, 0],
                                        preferred_element_type=jnp.float32)
                    acc = acc + jnp.dot(x2b, w12_ref[k, 1],
                                        preferred_element_type=jnp.float32)
                    return acc

                hxv = (hx0_ref if wi == 0 else hx1_ref)[sl, :]
                if gate:
                    r = jax.nn.sigmoid(cmb(3 * wi))
                    rh_s[sl, :] = r * hxv
                    u_s[sl, :] = jax.nn.sigmoid(cmb(3 * wi + 1))
                else:
                    c = jnp.tanh(cmb(3 * wi + 2))
                    u = u_s[sl, :]
                    nh = nh0_ref if wi == 0 else nh1_ref
                    nh[sl, :] = u * hxv + (1.0 - u) * c
                return 0

            jax.lax.fori_loop(0, NP, x2_body, 0)

        def cur0(sl):
            return inp_ref[:, sl].T

        def hx0(sl):
            return hx0_ref[sl, :]

        def rh(sl):
            return rh_s[sl, :]

        gconv(cur0, hx0, 1, 0, True)
        gconv(cur0, rh, 1, 0, False)

        def cur1(sl):
            return nh0_ref[sl, :]

        def hx1(sl):
            return hx1_ref[sl, :]

        gconv(cur1, hx1, UNITS, 1, True)
        gconv(cur1, rh, UNITS, 1, False)


def _mega(adj, inputs, hx0_n, hx1_n, w0, w12, bias):
    build_idx = lambda t: (jnp.minimum(t, NJT * NJT - 1) // NJT,
                           jnp.minimum(t, NJT * NJT - 1) % NJT)
    build_idx_t = lambda t: (jnp.minimum(t, NJT * NJT - 1) % NJT,
                             jnp.minimum(t, NJT * NJT - 1) // NJT)
    return pl.pallas_call(
        _mega_body,
        grid=(NSTEPS,),
        in_specs=[
            pl.BlockSpec((BT, BT), build_idx),
            pl.BlockSpec((BT, BT), build_idx_t),
            pl.BlockSpec((B, N), lambda t: (0, 0)),
            pl.BlockSpec((N, BU), lambda t: (0, 0)),
            pl.BlockSpec((N, BU), lambda t: (0, 0)),
            pl.BlockSpec((6, F, BU), lambda t: (0, 0, 0)),
            pl.BlockSpec((6, 2, F, BU), lambda t: (0, 0, 0, 0)),
            pl.BlockSpec((6, BU), lambda t: (0, 0)),
        ],
        out_specs=[
            pl.BlockSpec((N, BU), lambda t: (0, 0)),
            pl.BlockSpec((N, BU), lambda t: (0, 0)),
        ],
        out_shape=[
            jax.ShapeDtypeStruct((N, BU), jnp.float32),
            jax.ShapeDtypeStruct((N, BU), jnp.float32),
        ],
        scratch_shapes=[
            pltpu.VMEM((NJT, N, BT), jnp.bfloat16),
            pltpu.VMEM((NJT, 1, BT), jnp.float32),
            pltpu.VMEM((1, BT), jnp.float32),
            pltpu.VMEM((N, F), jnp.float32),
            pltpu.VMEM((N, F), jnp.bfloat16),
            pltpu.VMEM((N, F), jnp.bfloat16),
            pltpu.VMEM((N, BU), jnp.float32),
            pltpu.VMEM((N, BU), jnp.float32),
        ],
    )(adj, adj, inputs, hx0_n, hx1_n, w0, w12, bias)


def _prep_w(W, C, O):
    Wr = jnp.transpose(W.reshape(C, M, O), (1, 0, 2))
    return jnp.pad(Wr, ((0, 0), (0, CPAD - C), (0, 0)))


def kernel(inputs, hidden_state, adj, W0_gate, b0_gate, W0_cand, b0_cand,
           W1_gate, b1_gate, W1_cand, b1_cand):
    eye = jnp.eye(B, dtype=jnp.float32)
    w0_list, w12_list, b_list = [], [], []
    for (Wg, bg, Wc, bc, C) in [
        (W0_gate, b0_gate, W0_cand, b0_cand, 1 + UNITS),
        (W1_gate, b1_gate, W1_cand, b1_cand, 2 * UNITS),
    ]:
        wg = _prep_w(Wg, C, 2 * UNITS)
        wc = _prep_w(Wc, C, UNITS)
        for wm, bv in ((wg[:, :, :UNITS], bg[:UNITS]),
                       (wg[:, :, UNITS:], bg[UNITS:]),
                       (wc, bc)):
            bd = jnp.stack([jnp.kron(eye, wm[m]) for m in range(M)])
            w0_list.append(bd[0])
            w12_list.append(bd[1:])
            b_list.append(jnp.tile(bv, B))
    w0 = jnp.stack(w0_list)
    w12 = jnp.stack(w12_list).astype(jnp.bfloat16)
    bias = jnp.stack(b_list)

    hx0_n = _h2n(hidden_state[0].reshape(B, N, UNITS))
    hx1_n = _h2n(hidden_state[1].reshape(B, N, UNITS))

    nh0_n, nh1_n = _mega(adj, inputs, hx0_n, hx1_n, w0, w12, bias)

    h0 = _n2b(nh0_n).reshape(B, N * UNITS)
    h1 = _n2b(nh1_n).reshape(B, N * UNITS)
    return h1, jnp.stack([h0, h1], axis=0)


# speedup vs baseline: 1.1151x; 1.1151x over previous
import jax
import jax.numpy as jnp
from jax.experimental import pallas as pl
from jax.experimental.pallas import tpu as pltpu

N = 4096
B = 8
UNITS = 16
BU = B * UNITS
M = 3
CPAD = 32
F = B * CPAD
BLK = 512
NJB = N // BLK
BT = 256
NJT = N // BT
BTR = 512
NTR = N // BTR
PBLK = 512
NP = N // PBLK
NSTEPS = NTR * NJT + 1


def _h2n_body(h_ref, o_ref):
    o_ref[...] = jnp.concatenate([h_ref[b] for b in range(B)], axis=1)


def _h2n(h_bnu):
    return pl.pallas_call(
        _h2n_body,
        grid=(NJB,),
        in_specs=[pl.BlockSpec((B, BLK, UNITS), lambda j: (0, j, 0))],
        out_specs=pl.BlockSpec((BLK, BU), lambda j: (j, 0)),
        out_shape=jax.ShapeDtypeStruct((N, BU), jnp.float32),
    )(h_bnu)


def _n2b_body(x_ref, o_ref):
    for b in range(B):
        o_ref[b] = x_ref[:, b * UNITS:(b + 1) * UNITS]


def _n2b(x_n):
    return pl.pallas_call(
        _n2b_body,
        grid=(NJB,),
        in_specs=[pl.BlockSpec((BLK, BU), lambda j: (j, 0))],
        out_specs=pl.BlockSpec((B, BLK, UNITS), lambda j: (0, j, 0)),
        out_shape=jax.ShapeDtypeStruct((B, N, UNITS), jnp.float32),
    )(x_n)


def _mega_body(a_ref, at_ref, inp_ref, hx0_ref, hx1_ref,
               w0_ref, w12_ref, bias_ref,
               nh0_ref, nh1_ref,
               amax_s, dis_s, accd, x0s, x1b_s, xs_s, rh_s, u_s):
    t = pl.program_id(0)

    @pl.when(t < NTR * NJT)
    def _():
        i = t // NJT
        j = t % NJT
        m = jnp.maximum(a_ref[...], at_ref[...].T)
        amax_s[j, pl.ds(i * BTR, BTR), :] = m.astype(jnp.bfloat16)

        @pl.when(j == 0)
        def _():
            accd[...] = jnp.zeros_like(accd)

        accd[...] += jnp.sum(m, axis=1, keepdims=True).T

        @pl.when(j == NJT - 1)
        def _():
            d = accd[...]
            dis_s[i] = jnp.where(
                d > 0, jax.lax.rsqrt(jnp.maximum(d, 1e-12)), 0.0)

    @pl.when(t == NTR * NJT)
    def _():
        def dcol(i):
            return dis_s[i].T

        def panel_dot(sl):
            def jbody(j, acc):
                return acc + jnp.dot(
                    amax_s[j, sl, :], xs_s[pl.ds(j * BT, BT), :],
                    preferred_element_type=jnp.float32)
            return jax.lax.fori_loop(
                0, NJT, jbody, jnp.zeros((PBLK, F), jnp.float32))

        def assemble(get_cur, get_h, ci):
            pad = CPAD - ci - UNITS

            def body(i, _):
                sl = pl.ds(i * BLK, BLK)
                curb = get_cur(sl)
                hb = get_h(sl)
                pieces = []
                for b in range(B):
                    sub = [curb[:, b * ci:(b + 1) * ci],
                           hb[:, b * UNITS:(b + 1) * UNITS]]
                    if pad:
                        sub.append(jnp.zeros((BLK, pad), jnp.float32))
                    pieces.append(jnp.concatenate(sub, axis=1))
                x0s[sl, :] = jnp.concatenate(pieces, axis=1)
                return 0

            jax.lax.fori_loop(0, NJB, body, 0)

        def scale_to_xs(src):
            def body(i, _):
                sl = pl.ds(i * BLK, BLK)
                xs_s[sl, :] = (src[sl, :] * dcol(i)).astype(jnp.bfloat16)
                return 0

            jax.lax.fori_loop(0, NJB, body, 0)

        def gconv(get_cur, get_h, ci, wi, gate):
            assemble(get_cur, get_h, ci)
            scale_to_xs(x0s)

            def x1_body(p, _):
                sl = pl.ds(p * PBLK, PBLK)
                x1b_s[sl, :] = (-dcol(p) * panel_dot(sl)
                                ).astype(jnp.bfloat16)
                return 0

            jax.lax.fori_loop(0, NP, x1_body, 0)
            scale_to_xs(x1b_s)

            def x2_body(p, _):
                sl = pl.ds(p * PBLK, PBLK)
                x2v = -2.0 * dcol(p) * panel_dot(sl) - x0s[sl, :]
                x2b = x2v.astype(jnp.bfloat16)

                def cmb(k):
                    acc = bias_ref[k][None, :]
                    acc = acc + jnp.dot(x0s[sl, :], w0_ref[k],
                                        preferred_element_type=jnp.float32)
                    acc = acc + jnp.dot(x1b_s[sl, :], w12_ref[k, 0],
                                        preferred_element_type=jnp.float32)
                    acc = acc + jnp.dot(x2b, w12_ref[k, 1],
                                        preferred_element_type=jnp.float32)
                    return acc

                hxv = (hx0_ref if wi == 0 else hx1_ref)[sl, :]
                if gate:
                    r = jax.nn.sigmoid(cmb(3 * wi))
                    rh_s[sl, :] = (r * hxv).astype(jnp.bfloat16)
                    u_s[sl, :] = jax.nn.sigmoid(cmb(3 * wi + 1))
                else:
                    c = jnp.tanh(cmb(3 * wi + 2))
                    u = u_s[sl, :]
                    nh = nh0_ref if wi == 0 else nh1_ref
                    nh[sl, :] = u * hxv + (1.0 - u) * c
                return 0

            jax.lax.fori_loop(0, NP, x2_body, 0)

        def cur0(sl):
            return inp_ref[:, sl].T

        def hx0(sl):
            return hx0_ref[sl, :]

        def rh(sl):
            return rh_s[sl, :].astype(jnp.float32)

        gconv(cur0, hx0, 1, 0, True)
        gconv(cur0, rh, 1, 0, False)

        def cur1(sl):
            return nh0_ref[sl, :]

        def hx1(sl):
            return hx1_ref[sl, :]

        gconv(cur1, hx1, UNITS, 1, True)
        gconv(cur1, rh, UNITS, 1, False)


def _mega(adj, inputs, hx0_n, hx1_n, w0, w12, bias):
    build_idx = lambda t: (jnp.minimum(t, NTR * NJT - 1) // NJT,
                           jnp.minimum(t, NTR * NJT - 1) % NJT)
    build_idx_t = lambda t: (jnp.minimum(t, NTR * NJT - 1) % NJT,
                             jnp.minimum(t, NTR * NJT - 1) // NJT)
    return pl.pallas_call(
        _mega_body,
        grid=(NSTEPS,),
        in_specs=[
            pl.BlockSpec((BTR, BT), build_idx),
            pl.BlockSpec((BT, BTR), build_idx_t),
            pl.BlockSpec((B, N), lambda t: (0, 0)),
            pl.BlockSpec((N, BU), lambda t: (0, 0)),
            pl.BlockSpec((N, BU), lambda t: (0, 0)),
            pl.BlockSpec((6, F, BU), lambda t: (0, 0, 0)),
            pl.BlockSpec((6, 2, F, BU), lambda t: (0, 0, 0, 0)),
            pl.BlockSpec((6, BU), lambda t: (0, 0)),
        ],
        out_specs=[
            pl.BlockSpec((N, BU), lambda t: (0, 0)),
            pl.BlockSpec((N, BU), lambda t: (0, 0)),
        ],
        out_shape=[
            jax.ShapeDtypeStruct((N, BU), jnp.float32),
            jax.ShapeDtypeStruct((N, BU), jnp.float32),
        ],
        scratch_shapes=[
            pltpu.VMEM((NJT, N, BT), jnp.bfloat16),
            pltpu.VMEM((NTR, 1, BTR), jnp.float32),
            pltpu.VMEM((1, BTR), jnp.float32),
            pltpu.VMEM((N, F), jnp.float32),
            pltpu.VMEM((N, F), jnp.bfloat16),
            pltpu.VMEM((N, F), jnp.bfloat16),
            pltpu.VMEM((N, BU), jnp.bfloat16),
            pltpu.VMEM((N, BU), jnp.float32),
        ],
    )(adj, adj, inputs, hx0_n, hx1_n, w0, w12, bias)


def _prep_w(W, C, O):
    Wr = jnp.transpose(W.reshape(C, M, O), (1, 0, 2))
    return jnp.pad(Wr, ((0, 0), (0, CPAD - C), (0, 0)))


def kernel(inputs, hidden_state, adj, W0_gate, b0_gate, W0_cand, b0_cand,
           W1_gate, b1_gate, W1_cand, b1_cand):
    eye = jnp.eye(B, dtype=jnp.float32)
    w0_list, w12_list, b_list = [], [], []
    for (Wg, bg, Wc, bc, C) in [
        (W0_gate, b0_gate, W0_cand, b0_cand, 1 + UNITS),
        (W1_gate, b1_gate, W1_cand, b1_cand, 2 * UNITS),
    ]:
        wg = _prep_w(Wg, C, 2 * UNITS)
        wc = _prep_w(Wc, C, UNITS)
        for wm, bv in ((wg[:, :, :UNITS], bg[:UNITS]),
                       (wg[:, :, UNITS:], bg[UNITS:]),
                       (wc, bc)):
            bd = jnp.stack([jnp.kron(eye, wm[m]) for m in range(M)])
            w0_list.append(bd[0])
            w12_list.append(bd[1:])
            b_list.append(jnp.tile(bv, B))
    w0 = jnp.stack(w0_list)
    w12 = jnp.stack(w12_list).astype(jnp.bfloat16)
    bias = jnp.stack(b_list)

    hx0_n = _h2n(hidden_state[0].reshape(B, N, UNITS))
    hx1_n = _h2n(hidden_state[1].reshape(B, N, UNITS))

    nh0_n, nh1_n = _mega(adj, inputs, hx0_n, hx1_n, w0, w12, bias)

    h0 = _n2b(nh0_n).reshape(B, N * UNITS)
    h1 = _n2b(nh1_n).reshape(B, N * UNITS)
    return h1, jnp.stack([h0, h1], axis=0)


# speedup vs baseline: 1.8308x; 1.6418x over previous
import jax
import jax.numpy as jnp
from jax.experimental import pallas as pl
from jax.experimental.pallas import tpu as pltpu

N = 4096
B = 8
UNITS = 16
BU = B * UNITS
M = 3
CPAD = 32
F = B * CPAD
BLK = 512
NJB = N // BLK
PBLK = 512
NP = N // PBLK
NSTEPS = NJB * NJB + 1


def _h2n_body(h_ref, o_ref):
    o_ref[...] = jnp.concatenate([h_ref[b] for b in range(B)], axis=1)


def _h2n(h_bnu):
    return pl.pallas_call(
        _h2n_body,
        grid=(NJB,),
        in_specs=[pl.BlockSpec((B, BLK, UNITS), lambda j: (0, j, 0))],
        out_specs=pl.BlockSpec((BLK, BU), lambda j: (j, 0)),
        out_shape=jax.ShapeDtypeStruct((N, BU), jnp.float32),
    )(h_bnu)


def _n2b_body(x_ref, o_ref):
    for b in range(B):
        o_ref[b] = x_ref[:, b * UNITS:(b + 1) * UNITS]


def _n2b(x_n):
    return pl.pallas_call(
        _n2b_body,
        grid=(NJB,),
        in_specs=[pl.BlockSpec((BLK, BU), lambda j: (j, 0))],
        out_specs=pl.BlockSpec((B, BLK, UNITS), lambda j: (0, j, 0)),
        out_shape=jax.ShapeDtypeStruct((B, N, UNITS), jnp.float32),
    )(x_n)


def _mega_body(a_ref, at_ref, inp_ref, hx0_ref, hx1_ref,
               w_ref, bias_ref,
               nh0_ref, nh1_ref,
               amax_s, dis_s, accd, x0s, x1b_s, xs_s, rh_s, u_s):
    t = pl.program_id(0)

    @pl.when(t < NJB * NJB)
    def _():
        i = t // NJB
        j = t % NJB
        m = (jnp.maximum(a_ref[...], at_ref[...].T)).astype(jnp.bfloat16)
        for k in range(NJB):
            @pl.when(j == k)
            def _(k=k):
                amax_s[pl.ds(i * BLK, BLK), k * BLK:(k + 1) * BLK] = m

        mf = m.astype(jnp.float32)

        @pl.when(j == 0)
        def _():
            accd[...] = jnp.zeros_like(accd)

        accd[...] += jnp.sum(mf, axis=1, keepdims=True).T

        @pl.when(j == NJB - 1)
        def _():
            d = accd[...]
            dis_s[i] = jnp.where(
                d > 0, jax.lax.rsqrt(jnp.maximum(d, 1e-12)), 0.0)

    @pl.when(t == NJB * NJB)
    def _():
        def dcol(i):
            return dis_s[i].T

        def panel_dot(sl):
            return jnp.dot(amax_s[sl, :], xs_s[...],
                           preferred_element_type=jnp.float32)

        def assemble(get_cur, get_h, ci):
            pad = CPAD - ci - UNITS

            def body(i, _):
                sl = pl.ds(i * BLK, BLK)
                curb = get_cur(sl)
                hb = get_h(sl)
                pieces = []
                for b in range(B):
                    sub = [curb[:, b * ci:(b + 1) * ci],
                           hb[:, b * UNITS:(b + 1) * UNITS]]
                    if pad:
                        sub.append(jnp.zeros((BLK, pad), jnp.float32))
                    pieces.append(jnp.concatenate(sub, axis=1))
                x0s[sl, :] = jnp.concatenate(
                    pieces, axis=1).astype(jnp.bfloat16)
                return 0

            jax.lax.fori_loop(0, NJB, body, 0)

        def scale_to_xs(src):
            def body(i, _):
                sl = pl.ds(i * BLK, BLK)
                xs_s[sl, :] = (src[sl, :] * dcol(i)).astype(jnp.bfloat16)
                return 0

            jax.lax.fori_loop(0, NJB, body, 0)

        def gconv(get_cur, get_h, ci, wi, gate):
            assemble(get_cur, get_h, ci)
            scale_to_xs(x0s)

            def x1_body(p, _):
                sl = pl.ds(p * PBLK, PBLK)
                x1b_s[sl, :] = (-dcol(p) * panel_dot(sl)
                                ).astype(jnp.bfloat16)
                return 0

            jax.lax.fori_loop(0, NP, x1_body, 0)
            scale_to_xs(x1b_s)

            def x2_body(p, _):
                sl = pl.ds(p * PBLK, PBLK)
                x2v = (-2.0 * dcol(p) * panel_dot(sl)
                       - x0s[sl, :].astype(jnp.float32))
                x2b = x2v.astype(jnp.bfloat16)

                def cmb(k):
                    acc = bias_ref[k][None, :]
                    acc = acc + jnp.dot(x0s[sl, :], w_ref[k, 0],
                                        preferred_element_type=jnp.float32)
                    acc = acc + jnp.dot(x1b_s[sl, :], w_ref[k, 1],
                                        preferred_element_type=jnp.float32)
                    acc = acc + jnp.dot(x2b, w_ref[k, 2],
                                        preferred_element_type=jnp.float32)
                    return acc

                hxv = (hx0_ref if wi == 0 else hx1_ref)[sl, :]
                if gate:
                    r = jax.nn.sigmoid(cmb(3 * wi))
                    rh_s[sl, :] = (r * hxv).astype(jnp.bfloat16)
                    u_s[sl, :] = jax.nn.sigmoid(
                        cmb(3 * wi + 1)).astype(jnp.bfloat16)
                else:
                    c = jnp.tanh(cmb(3 * wi + 2))
                    u = u_s[sl, :].astype(jnp.float32)
                    nh = nh0_ref if wi == 0 else nh1_ref
                    nh[sl, :] = u * hxv + (1.0 - u) * c
                return 0

            jax.lax.fori_loop(0, NP, x2_body, 0)

        def cur0(sl):
            return inp_ref[:, sl].T

        def hx0(sl):
            return hx0_ref[sl, :]

        def rh(sl):
            return rh_s[sl, :].astype(jnp.float32)

        gconv(cur0, hx0, 1, 0, True)
        gconv(cur0, rh, 1, 0, False)

        def cur1(sl):
            return nh0_ref[sl, :]

        def hx1(sl):
            return hx1_ref[sl, :]

        gconv(cur1, hx1, UNITS, 1, True)
        gconv(cur1, rh, UNITS, 1, False)


def _mega(adj, inputs, hx0_n, hx1_n, w_all, bias):
    build_idx = lambda t: (jnp.minimum(t, NJB * NJB - 1) // NJB,
                           jnp.minimum(t, NJB * NJB - 1) % NJB)
    build_idx_t = lambda t: (jnp.minimum(t, NJB * NJB - 1) % NJB,
                             jnp.minimum(t, NJB * NJB - 1) // NJB)
    return pl.pallas_call(
        _mega_body,
        grid=(NSTEPS,),
        in_specs=[
            pl.BlockSpec((BLK, BLK), build_idx),
            pl.BlockSpec((BLK, BLK), build_idx_t),
            pl.BlockSpec((B, N), lambda t: (0, 0)),
            pl.BlockSpec((N, BU), lambda t: (0, 0)),
            pl.BlockSpec((N, BU), lambda t: (0, 0)),
            pl.BlockSpec((6, 3, F, BU), lambda t: (0, 0, 0, 0)),
            pl.BlockSpec((6, BU), lambda t: (0, 0)),
        ],
        out_specs=[
            pl.BlockSpec((N, BU), lambda t: (0, 0)),
            pl.BlockSpec((N, BU), lambda t: (0, 0)),
        ],
        out_shape=[
            jax.ShapeDtypeStruct((N, BU), jnp.float32),
            jax.ShapeDtypeStruct((N, BU), jnp.float32),
        ],
        scratch_shapes=[
            pltpu.VMEM((N, N), jnp.bfloat16),
            pltpu.VMEM((NJB, 1, BLK), jnp.float32),
            pltpu.VMEM((1, BLK), jnp.float32),
            pltpu.VMEM((N, F), jnp.bfloat16),
            pltpu.VMEM((N, F), jnp.bfloat16),
            pltpu.VMEM((N, F), jnp.bfloat16),
            pltpu.VMEM((N, BU), jnp.bfloat16),
            pltpu.VMEM((N, BU), jnp.bfloat16),
        ],
    )(adj, adj, inputs, hx0_n, hx1_n, w_all, bias)


def _prep_w(W, C, O):
    Wr = jnp.transpose(W.reshape(C, M, O), (1, 0, 2))
    return jnp.pad(Wr, ((0, 0), (0, CPAD - C), (0, 0)))


def kernel(inputs, hidden_state, adj, W0_gate, b0_gate, W0_cand, b0_cand,
           W1_gate, b1_gate, W1_cand, b1_cand):
    eye = jnp.eye(B, dtype=jnp.float32)
    w_list, b_list = [], []
    for (Wg, bg, Wc, bc, C) in [
        (W0_gate, b0_gate, W0_cand, b0_cand, 1 + UNITS),
        (W1_gate, b1_gate, W1_cand, b1_cand, 2 * UNITS),
    ]:
        wg = _prep_w(Wg, C, 2 * UNITS)
        wc = _prep_w(Wc, C, UNITS)
        for wm, bv in ((wg[:, :, :UNITS], bg[:UNITS]),
                       (wg[:, :, UNITS:], bg[UNITS:]),
                       (wc, bc)):
            bd = jnp.stack([jnp.kron(eye, wm[m]) for m in range(M)])
            w_list.append(bd)
            b_list.append(jnp.tile(bv, B))
    w_all = jnp.stack(w_list).astype(jnp.bfloat16)
    bias = jnp.stack(b_list)

    hx0_n = _h2n(hidden_state[0].reshape(B, N, UNITS))
    hx1_n = _h2n(hidden_state[1].reshape(B, N, UNITS))

    nh0_n, nh1_n = _mega(adj, inputs, hx0_n, hx1_n, w_all, bias)

    h0 = _n2b(nh0_n).reshape(B, N * UNITS)
    h1 = _n2b(nh1_n).reshape(B, N * UNITS)
    return h1, jnp.stack([h0, h1], axis=0)
